# Initial kernel scaffold; baseline (speedup 1.0000x reference)
import jax
import jax.numpy as jnp
from jax.experimental import pallas as pl


def kernel(x, edge_index, edge_weight, pos, W1, b1, W2, b2):
    raise NotImplementedError("write your pallas kernel here")


# speedup vs baseline: 1.6567x; 1.6567x over previous
import functools

import jax
import jax.numpy as jnp
from jax import lax
from jax.experimental import pallas as pl
from jax.experimental.pallas import tpu as pltpu
from jax.experimental.pallas import tpu_sc as plsc

N = 10000
E = 320000
D = 128
POS = 8
NT = 16
EB = 128
NBLK = 2560
PE = NBLK * EB
BPT = NBLK // NT
NP = 10240
NPASS = 4
HNP = NP // NPASS
HPT = HNP // NT
HCH = 32
NCH = HPT // HCH

_MESH = plsc.VectorSubcoreMesh(core_axis_name="c", subcore_axis_name="s",
                               num_cores=1)
_CPARAMS = pltpu.CompilerParams(needs_layout_passes=False)


MASK14 = 16383


@functools.partial(
    pl.kernel, mesh=_MESH, compiler_params=_CPARAMS,
    out_type=jax.ShapeDtypeStruct((NT, NP), jnp.float32),
    scratch_types=[
        pltpu.VMEM((8, EB), jnp.int32),
        pltpu.VMEM((8, EB), jnp.float32),
        pltpu.VMEM((NP,), jnp.float32),
    ],
)
def _deg_kernel(comb_hbm, ws_hbm, deg_hbm, cch, wch, degb):
    wid = lax.axis_index("s")

    def zero_body(i, _):
        degb[pl.ds(i * 16, 16)] = jnp.zeros((16,), jnp.float32)
        return 0
    lax.fori_loop(0, NP // 16, zero_body, 0)

    def chunk_body(c, _):
        pltpu.sync_copy(comb_hbm.at[pl.ds(wid * BPT + c * 8, 8)], cch)
        pltpu.sync_copy(ws_hbm.at[pl.ds(wid * BPT + c * 8, 8)], wch)
        for b in range(8):
            for g in range(EB // 16):
                sl = pl.ds(g * 16, 16)
                di = lax.shift_right_logical(cch[b, sl], 14)
                plsc.addupdate_scatter(degb, [di], wch[b, sl])
        return 0
    lax.fori_loop(0, BPT // 8, chunk_body, 0)
    pltpu.sync_copy(degb, deg_hbm.at[wid])


def _dinv_tc(deg_parts):
    def body(d_ref, o_ref):
        o_ref[...] = lax.rsqrt(jnp.sum(d_ref[...], axis=0) + 1.0)
    return pl.pallas_call(
        body,
        out_shape=jax.ShapeDtypeStruct((NP // D, D), jnp.float32),
    )(deg_parts)


def _mm_bias_tc(x, w, b, relu, bm):
    m = x.shape[0]

    def body(x_ref, w_ref, b_ref, o_ref):
        v = jnp.dot(x_ref[...], w_ref[...],
                    preferred_element_type=jnp.float32) + b_ref[...]
        if relu:
            v = jnp.maximum(v, 0.0)
        o_ref[...] = v

    return pl.pallas_call(
        body,
        grid=(m // bm,),
        in_specs=[pl.BlockSpec((bm, D), lambda i: (i, 0)),
                  pl.BlockSpec((D, D), lambda i: (0, 0)),
                  pl.BlockSpec((1, D), lambda i: (0, 0))],
        out_specs=pl.BlockSpec((bm, D), lambda i: (i, 0)),
        out_shape=jax.ShapeDtypeStruct((m, D), jnp.float32),
    )(x, w, b.reshape(1, D))


@functools.partial(
    pl.kernel, mesh=_MESH, compiler_params=_CPARAMS,
    out_type=jax.ShapeDtypeStruct((NP, D), jnp.float32),
    scratch_types=[
        pltpu.VMEM((BPT, EB), jnp.int32),
        pltpu.VMEM((BPT, EB), jnp.float32),
        pltpu.VMEM((NP,), jnp.float32),
        pltpu.VMEM((EB,), jnp.int32),
        pltpu.VMEM((EB,), jnp.int32),
        pltpu.VMEM((EB,), jnp.float32),
        pltpu.VMEM((EB, D), jnp.float32),
        pltpu.VMEM((HCH, D), jnp.float32),
        pltpu.VMEM((HCH, D), jnp.float32),
        pltpu.VMEM_SHARED((HNP, D), jnp.float32),
        pltpu.SemaphoreType.DMA,
    ],
)
def _agg_kernel(comb_hbm, ws_hbm, dinv_hbm, x_hbm, o_hbm,
                combb, wb, dinvb, sblk, dstadj, coefp, rows, nbuf, zbuf,
                acc, sem):
    wid = lax.axis_index("s")

    def zb_body(i, _):
        for j in range(D // 16):
            zbuf[i, pl.ds(j * 16, 16)] = jnp.zeros((16,), jnp.float32)
        return 0
    lax.fori_loop(0, HCH, zb_body, 0)

    pltpu.sync_copy(comb_hbm.at[pl.ds(wid * BPT, BPT)], combb)
    pltpu.sync_copy(ws_hbm.at[pl.ds(wid * BPT, BPT)], wb)
    pltpu.sync_copy(dinv_hbm, dinvb)

    def pass_body(p, _):
        lo = p * HNP

        for c in range(NCH):
            pltpu.sync_copy(zbuf, acc.at[pl.ds(wid * HPT + c * HCH, HCH)])
        plsc.subcore_barrier()

        def block_body(b, _):
            for g in range(EB // 16):
                sl = pl.ds(g * 16, 16)
                cv = combb[b, sl]
                si = cv & MASK14
                di = lax.shift_right_logical(cv, 14)
                wv = wb[b, sl]
                da = plsc.load_gather(dinvb, [si])
                db = plsc.load_gather(dinvb, [di])
                dl = di - lo
                inr = (dl >= 0) & (dl < HNP)
                coefp[sl] = jnp.where(inr, da * db * wv, 0.0)
                dstadj[sl] = jnp.clip(dl, 0, HNP - 1)
                sblk[sl] = si
            pltpu.async_copy(x_hbm.at[sblk], rows, sem).wait()

            def scale_body(g, _):
                c16 = coefp[pl.ds(g * 16, 16)]
                for k in range(16):
                    cc = c16[k]
                    e = g * 16 + k
                    for j in range(D // 16):
                        sl = pl.ds(j * 16, 16)
                        rows[e, sl] = rows[e, sl] * cc
                return 0
            lax.fori_loop(0, EB // 16, scale_body, 0)
            pltpu.sync_copy(rows, acc.at[dstadj], add=True)
            return 0
        lax.fori_loop(0, BPT, block_body, 0)
        plsc.subcore_barrier()

        for c in range(NCH):
            lbase = wid * HPT + c * HCH
            gbase = lo + lbase
            pltpu.sync_copy(acc.at[pl.ds(lbase, HCH)],
                            rows.at[pl.ds(0, HCH)])
            pltpu.sync_copy(x_hbm.at[pl.ds(gbase, HCH)], nbuf)

            def node_body(g, _):
                dv16 = dinvb[pl.ds(gbase + g * 16, 16)]
                for k in range(16):
                    dv = dv16[k]
                    dv2 = dv * dv
                    nn = g * 16 + k
                    for j in range(D // 16):
                        sl = pl.ds(j * 16, 16)
                        rows[nn, sl] = rows[nn, sl] + dv2 * nbuf[nn, sl]
                return 0
            lax.fori_loop(0, HCH // 16, node_body, 0)
            pltpu.sync_copy(rows.at[pl.ds(0, HCH)],
                            o_hbm.at[pl.ds(gbase, HCH)])
        plsc.subcore_barrier()
        return 0
    lax.fori_loop(0, NPASS, pass_body, 0)


@functools.partial(
    pl.kernel, mesh=_MESH, compiler_params=_CPARAMS,
    out_type=jax.ShapeDtypeStruct((2 * POS, D), jnp.float32),
    scratch_types=[
        pltpu.VMEM((2 * POS,), jnp.int32),
        pltpu.VMEM((2 * POS, D), jnp.float32),
        pltpu.SemaphoreType.DMA,
    ],
)
def _gather_pos_kernel(a_hbm, pos_hbm, g_hbm, posb, buf, sem):
    wid = lax.axis_index("s")

    @pl.when(wid == 0)
    def _():
        pltpu.sync_copy(pos_hbm, posb)
        pltpu.async_copy(a_hbm.at[posb], buf, sem).wait()
        pltpu.sync_copy(buf, g_hbm)


def _final_mm_tc(pg, w, b):
    def body(x_ref, w_ref, b_ref, o_ref):
        o_ref[...] = jnp.dot(x_ref[...], w_ref[...],
                             preferred_element_type=jnp.float32) + b_ref[...]
    return pl.pallas_call(
        body,
        out_shape=jax.ShapeDtypeStruct((2 * POS, D), jnp.float32),
    )(pg, w, b.reshape(1, D))


@jax.jit
def kernel(x, edge_index, edge_weight, pos, W1, b1, W2, b2):
    pad = PE - E
    comb = edge_index[0] | (edge_index[1] << 14)
    combp = jnp.concatenate(
        [comb, jnp.zeros((pad,), jnp.int32)]).reshape(NBLK, EB)
    wp = jnp.concatenate(
        [edge_weight, jnp.zeros((pad,), jnp.float32)]).reshape(NBLK, EB)

    deg_parts = _deg_kernel(combp, wp)
    dinv = _dinv_tc(deg_parts.reshape(NT, NP // D, D)).reshape(NP)

    xp = jnp.concatenate([x, jnp.zeros((NP - N, D), jnp.float32)])
    a1 = _agg_kernel(combp, wp, dinv, xp)
    h = _mm_bias_tc(a1, W1, b1, relu=True, bm=512)
    a2 = _agg_kernel(combp, wp, dinv, h)

    posp = jnp.concatenate([pos, jnp.zeros((POS,), jnp.int32)])
    pg = _gather_pos_kernel(a2, posp)
    out16 = _final_mm_tc(pg, W2, b2)

    out = jnp.where((pos != -1)[:, None], out16[:POS], jnp.float32(-3.0))
    return out.reshape(1, POS * D)


# speedup vs baseline: 1.9256x; 1.1623x over previous
import functools

import jax
import jax.numpy as jnp
from jax import lax
from jax.experimental import pallas as pl
from jax.experimental.pallas import tpu as pltpu
from jax.experimental.pallas import tpu_sc as plsc

N = 10000
E = 320000
D = 128
POS = 8
NC = 2
NT = 16
NW = NC * NT
EB = 128
NBLK = 2560
PE = NBLK * EB
BPT = NBLK // NW
NP = 10240
NPASS = 4
HNP = NP // NPASS
HPT = HNP // NT
HCH = 32
NCH = HPT // HCH

_MESH = plsc.VectorSubcoreMesh(core_axis_name="c", subcore_axis_name="s",
                               num_cores=NC)
_CPARAMS = pltpu.CompilerParams(needs_layout_passes=False)


MASK14 = 16383


@functools.partial(
    pl.kernel, mesh=_MESH, compiler_params=_CPARAMS,
    out_type=jax.ShapeDtypeStruct((NW, NP), jnp.float32),
    scratch_types=[
        pltpu.VMEM((8, EB), jnp.int32),
        pltpu.VMEM((8, EB), jnp.float32),
        pltpu.VMEM((NP,), jnp.float32),
    ],
)
def _deg_kernel(comb_hbm, ws_hbm, deg_hbm, cch, wch, degb):
    wid = lax.axis_index("c") * NT + lax.axis_index("s")

    def zero_body(i, _):
        degb[pl.ds(i * 16, 16)] = jnp.zeros((16,), jnp.float32)
        return 0
    lax.fori_loop(0, NP // 16, zero_body, 0)

    def chunk_body(c, _):
        pltpu.sync_copy(comb_hbm.at[pl.ds(wid * BPT + c * 8, 8)], cch)
        pltpu.sync_copy(ws_hbm.at[pl.ds(wid * BPT + c * 8, 8)], wch)
        for b in range(8):
            for g in range(EB // 16):
                sl = pl.ds(g * 16, 16)
                di = lax.shift_right_logical(cch[b, sl], 14)
                plsc.addupdate_scatter(degb, [di], wch[b, sl])
        return 0
    lax.fori_loop(0, BPT // 8, chunk_body, 0)
    pltpu.sync_copy(degb, deg_hbm.at[wid])


def _dinv_tc(deg_parts):
    def body(d_ref, o_ref):
        o_ref[...] = lax.rsqrt(jnp.sum(d_ref[...], axis=0) + 1.0)
    return pl.pallas_call(
        body,
        out_shape=jax.ShapeDtypeStruct((NP // D, D), jnp.float32),
    )(deg_parts)


def _mm_bias_tc(x0, x1, w, b, relu, bm):
    m = x0.shape[0]

    def body(x0_ref, x1_ref, w_ref, b_ref, o_ref):
        v = jnp.dot(x0_ref[...] + x1_ref[...], w_ref[...],
                    preferred_element_type=jnp.float32) + b_ref[...]
        if relu:
            v = jnp.maximum(v, 0.0)
        o_ref[...] = v

    return pl.pallas_call(
        body,
        grid=(m // bm,),
        in_specs=[pl.BlockSpec((bm, D), lambda i: (i, 0)),
                  pl.BlockSpec((bm, D), lambda i: (i, 0)),
                  pl.BlockSpec((D, D), lambda i: (0, 0)),
                  pl.BlockSpec((1, D), lambda i: (0, 0))],
        out_specs=pl.BlockSpec((bm, D), lambda i: (i, 0)),
        out_shape=jax.ShapeDtypeStruct((m, D), jnp.float32),
    )(x0, x1, w, b.reshape(1, D))


@functools.partial(
    pl.kernel, mesh=_MESH, compiler_params=_CPARAMS,
    out_type=(jax.ShapeDtypeStruct((NP, D), jnp.float32),
              jax.ShapeDtypeStruct((NP, D), jnp.float32)),
    scratch_types=[
        pltpu.VMEM((BPT, EB), jnp.int32),
        pltpu.VMEM((BPT, EB), jnp.float32),
        pltpu.VMEM((NP,), jnp.float32),
        pltpu.VMEM((EB,), jnp.int32),
        pltpu.VMEM((EB,), jnp.int32),
        pltpu.VMEM((EB,), jnp.float32),
        pltpu.VMEM((EB, D), jnp.float32),
        pltpu.VMEM((HCH, D), jnp.float32),
        pltpu.VMEM((HCH, D), jnp.float32),
        pltpu.VMEM_SHARED((HNP, D), jnp.float32),
        pltpu.SemaphoreType.DMA,
    ],
)
def _agg_kernel(comb_hbm, ws_hbm, dinv_hbm, x_hbm, o0_hbm, o1_hbm,
                combb, wb, dinvb, sblk, dstadj, coefp, rows, nbuf, zbuf,
                acc, sem):
    cid = lax.axis_index("c")
    wid = lax.axis_index("s")
    w_id = cid * NT + wid
    slw = jnp.where(cid == 0, 1.0, 0.0)

    def zb_body(i, _):
        for j in range(D // 16):
            zbuf[i, pl.ds(j * 16, 16)] = jnp.zeros((16,), jnp.float32)
        return 0
    lax.fori_loop(0, HCH, zb_body, 0)

    pltpu.sync_copy(comb_hbm.at[pl.ds(w_id * BPT, BPT)], combb)
    pltpu.sync_copy(ws_hbm.at[pl.ds(w_id * BPT, BPT)], wb)
    pltpu.sync_copy(dinv_hbm, dinvb)

    def pass_body(p, _):
        lo = p * HNP

        for c in range(NCH):
            pltpu.sync_copy(zbuf, acc.at[pl.ds(wid * HPT + c * HCH, HCH)])
        plsc.subcore_barrier()

        def block_body(b, _):
            for g in range(EB // 16):
                sl = pl.ds(g * 16, 16)
                cv = combb[b, sl]
                si = cv & MASK14
                di = lax.shift_right_logical(cv, 14)
                wv = wb[b, sl]
                da = plsc.load_gather(dinvb, [si])
                db = plsc.load_gather(dinvb, [di])
                dl = di - lo
                inr = (dl >= 0) & (dl < HNP)
                coefp[sl] = jnp.where(inr, da * db * wv, 0.0)
                dstadj[sl] = jnp.clip(dl, 0, HNP - 1)
                sblk[sl] = si
            pltpu.async_copy(x_hbm.at[sblk], rows, sem).wait()

            def scale_body(g, _):
                c16 = coefp[pl.ds(g * 16, 16)]
                for k in range(16):
                    cc = c16[k]
                    e = g * 16 + k
                    for j in range(D // 16):
                        sl = pl.ds(j * 16, 16)
                        rows[e, sl] = rows[e, sl] * cc
                return 0
            lax.fori_loop(0, EB // 16, scale_body, 0)
            pltpu.sync_copy(rows, acc.at[dstadj], add=True)
            return 0
        lax.fori_loop(0, BPT, block_body, 0)
        plsc.subcore_barrier()

        for c in range(NCH):
            lbase = wid * HPT + c * HCH
            gbase = lo + lbase
            pltpu.sync_copy(acc.at[pl.ds(lbase, HCH)],
                            rows.at[pl.ds(0, HCH)])
            pltpu.sync_copy(x_hbm.at[pl.ds(gbase, HCH)], nbuf)

            def node_body(g, _):
                dv16 = dinvb[pl.ds(gbase + g * 16, 16)]
                for k in range(16):
                    dv = dv16[k]
                    dv2 = dv * dv * slw
                    nn = g * 16 + k
                    for j in range(D // 16):
                        sl = pl.ds(j * 16, 16)
                        rows[nn, sl] = rows[nn, sl] + dv2 * nbuf[nn, sl]
                return 0
            lax.fori_loop(0, HCH // 16, node_body, 0)

            @pl.when(cid == 0)
            def _():
                pltpu.sync_copy(rows.at[pl.ds(0, HCH)],
                                o0_hbm.at[pl.ds(gbase, HCH)])

            @pl.when(cid == 1)
            def _():
                pltpu.sync_copy(rows.at[pl.ds(0, HCH)],
                                o1_hbm.at[pl.ds(gbase, HCH)])
        plsc.subcore_barrier()
        return 0
    lax.fori_loop(0, NPASS, pass_body, 0)


@functools.partial(
    pl.kernel, mesh=_MESH, compiler_params=_CPARAMS,
    out_type=(jax.ShapeDtypeStruct((2 * POS, D), jnp.float32),
              jax.ShapeDtypeStruct((2 * POS, D), jnp.float32)),
    scratch_types=[
        pltpu.VMEM((2 * POS,), jnp.int32),
        pltpu.VMEM((2 * POS, D), jnp.float32),
        pltpu.SemaphoreType.DMA,
    ],
)
def _gather_pos_kernel(a0_hbm, a1_hbm, pos_hbm, g0_hbm, g1_hbm,
                       posb, buf, sem):
    cid = lax.axis_index("c")
    wid = lax.axis_index("s")

    @pl.when((cid == 0) & (wid == 0))
    def _():
        pltpu.sync_copy(pos_hbm, posb)
        pltpu.async_copy(a0_hbm.at[posb], buf, sem).wait()
        pltpu.sync_copy(buf, g0_hbm)
        pltpu.async_copy(a1_hbm.at[posb], buf, sem).wait()
        pltpu.sync_copy(buf, g1_hbm)


def _final_mm_tc(pg0, pg1, w, b):
    def body(x0_ref, x1_ref, w_ref, b_ref, o_ref):
        o_ref[...] = jnp.dot(x0_ref[...] + x1_ref[...], w_ref[...],
                             preferred_element_type=jnp.float32) + b_ref[...]
    return pl.pallas_call(
        body,
        out_shape=jax.ShapeDtypeStruct((2 * POS, D), jnp.float32),
    )(pg0, pg1, w, b.reshape(1, D))


@jax.jit
def kernel(x, edge_index, edge_weight, pos, W1, b1, W2, b2):
    pad = PE - E
    comb = edge_index[0] | (edge_index[1] << 14)
    combp = jnp.concatenate(
        [comb, jnp.zeros((pad,), jnp.int32)]).reshape(NBLK, EB)
    wp = jnp.concatenate(
        [edge_weight, jnp.zeros((pad,), jnp.float32)]).reshape(NBLK, EB)

    deg_parts = _deg_kernel(combp, wp)
    dinv = _dinv_tc(deg_parts.reshape(NW, NP // D, D)).reshape(NP)

    xp = jnp.concatenate([x, jnp.zeros((NP - N, D), jnp.float32)])
    a1_0, a1_1 = _agg_kernel(combp, wp, dinv, xp)
    h = _mm_bias_tc(a1_0, a1_1, W1, b1, relu=True, bm=512)
    a2_0, a2_1 = _agg_kernel(combp, wp, dinv, h)

    posp = jnp.concatenate([pos, jnp.zeros((POS,), jnp.int32)])
    pg0, pg1 = _gather_pos_kernel(a2_0, a2_1, posp)
    out16 = _final_mm_tc(pg0, pg1, W2, b2)

    out = jnp.where((pos != -1)[:, None], out16[:POS], jnp.float32(-3.0))
    return out.reshape(1, POS * D)


# speedup vs baseline: 2.5817x; 1.3407x over previous
import functools

import jax
import jax.numpy as jnp
from jax import lax
from jax.experimental import pallas as pl
from jax.experimental.pallas import tpu as pltpu
from jax.experimental.pallas import tpu_sc as plsc

N = 10000
E = 320000
D = 128
POS = 8
NC = 2
NT = 16
NW = NC * NT
EB = 128
NBLK = 2560
PE = NBLK * EB
BPT = NBLK // NW
NP = 12288
NPASS = 3
HNP = NP // NPASS
HPT = HNP // NT
HCH = 32
NCH = HPT // HCH

_MESH = plsc.VectorSubcoreMesh(core_axis_name="c", subcore_axis_name="s",
                               num_cores=NC)
_CPARAMS = pltpu.CompilerParams(needs_layout_passes=False)


MASK14 = 16383


@functools.partial(
    pl.kernel, mesh=_MESH, compiler_params=_CPARAMS,
    out_type=jax.ShapeDtypeStruct((NW, NP), jnp.float32),
    scratch_types=[
        pltpu.VMEM((8, EB), jnp.int32),
        pltpu.VMEM((8, EB), jnp.float32),
        pltpu.VMEM((NP,), jnp.float32),
    ],
)
def _deg_kernel(comb_hbm, ws_hbm, deg_hbm, cch, wch, degb):
    wid = lax.axis_index("c") * NT + lax.axis_index("s")

    def zero_body(i, _):
        degb[pl.ds(i * 16, 16)] = jnp.zeros((16,), jnp.float32)
        return 0
    lax.fori_loop(0, NP // 16, zero_body, 0)

    def chunk_body(c, _):
        pltpu.sync_copy(comb_hbm.at[pl.ds(wid * BPT + c * 8, 8)], cch)
        pltpu.sync_copy(ws_hbm.at[pl.ds(wid * BPT + c * 8, 8)], wch)
        for b in range(8):
            for g in range(EB // 16):
                sl = pl.ds(g * 16, 16)
                di = lax.shift_right_logical(cch[b, sl], 14)
                plsc.addupdate_scatter(degb, [di], wch[b, sl])
        return 0
    lax.fori_loop(0, BPT // 8, chunk_body, 0)
    pltpu.sync_copy(degb, deg_hbm.at[wid])


def _dinv_tc(deg_parts):
    def body(d_ref, o_ref):
        o_ref[...] = lax.rsqrt(jnp.sum(d_ref[...], axis=0) + 1.0)
    return pl.pallas_call(
        body,
        out_shape=jax.ShapeDtypeStruct((NP // D, D), jnp.float32),
    )(deg_parts)


def _mm_bias_tc(x0, x1, w, b, relu, bm):
    m = x0.shape[0]

    def body(x0_ref, x1_ref, w_ref, b_ref, o_ref):
        v = jnp.dot(x0_ref[...] + x1_ref[...], w_ref[...],
                    preferred_element_type=jnp.float32) + b_ref[...]
        if relu:
            v = jnp.maximum(v, 0.0)
        o_ref[...] = v

    return pl.pallas_call(
        body,
        grid=(m // bm,),
        in_specs=[pl.BlockSpec((bm, D), lambda i: (i, 0)),
                  pl.BlockSpec((bm, D), lambda i: (i, 0)),
                  pl.BlockSpec((D, D), lambda i: (0, 0)),
                  pl.BlockSpec((1, D), lambda i: (0, 0))],
        out_specs=pl.BlockSpec((bm, D), lambda i: (i, 0)),
        out_shape=jax.ShapeDtypeStruct((m, D), jnp.float32),
    )(x0, x1, w, b.reshape(1, D))


@functools.partial(
    pl.kernel, mesh=_MESH, compiler_params=_CPARAMS,
    out_type=(jax.ShapeDtypeStruct((NP, D), jnp.float32),
              jax.ShapeDtypeStruct((NP, D), jnp.float32)),
    scratch_types=[
        pltpu.VMEM((BPT, EB), jnp.int32),
        pltpu.VMEM((BPT, EB), jnp.float32),
        pltpu.VMEM((NP,), jnp.float32),
        pltpu.VMEM((EB,), jnp.int32),
        pltpu.VMEM((EB,), jnp.int32),
        pltpu.VMEM((EB,), jnp.float32),
        pltpu.VMEM((EB, D), jnp.float32),
        pltpu.VMEM((HCH, D), jnp.float32),
        pltpu.VMEM((HCH, D), jnp.float32),
        pltpu.VMEM_SHARED((HNP, D), jnp.float32),
        pltpu.SemaphoreType.DMA,
    ],
)
def _agg_kernel(comb_hbm, ws_hbm, dinv_hbm, x_hbm, o0_hbm, o1_hbm,
                combb, wb, dinvb, sblk, dstadj, coefp, rows, nbuf, zbuf,
                acc, sem):
    cid = lax.axis_index("c")
    wid = lax.axis_index("s")
    w_id = cid * NT + wid
    slw = jnp.where(cid == 0, 1.0, 0.0)

    def zb_body(i, _):
        for j in range(D // 16):
            zbuf[i, pl.ds(j * 16, 16)] = jnp.zeros((16,), jnp.float32)
        return 0
    lax.fori_loop(0, HCH, zb_body, 0)

    pltpu.sync_copy(comb_hbm.at[pl.ds(w_id * BPT, BPT)], combb)
    pltpu.sync_copy(ws_hbm.at[pl.ds(w_id * BPT, BPT)], wb)
    pltpu.sync_copy(dinv_hbm, dinvb)

    def pass_body(p, _):
        lo = p * HNP

        for c in range(NCH):
            pltpu.sync_copy(zbuf, acc.at[pl.ds(wid * HPT + c * HCH, HCH)])
        plsc.subcore_barrier()

        def block_body(b, _):
            for g in range(EB // 16):
                sl = pl.ds(g * 16, 16)
                cv = combb[b, sl]
                si = cv & MASK14
                di = lax.shift_right_logical(cv, 14)
                wv = wb[b, sl]
                da = plsc.load_gather(dinvb, [si])
                db = plsc.load_gather(dinvb, [di])
                dl = di - lo
                inr = (dl >= 0) & (dl < HNP)
                coefp[sl] = jnp.where(inr, da * db * wv, 0.0)
                dstadj[sl] = jnp.clip(dl, 0, HNP - 1)
                sblk[sl] = si
            pltpu.async_copy(x_hbm.at[sblk], rows, sem).wait()

            def scale_body(g, _):
                c16 = coefp[pl.ds(g * 16, 16)]
                for k in range(16):
                    cc = c16[k]
                    e = g * 16 + k
                    for j in range(D // 16):
                        sl = pl.ds(j * 16, 16)
                        rows[e, sl] = rows[e, sl] * cc
                return 0
            lax.fori_loop(0, EB // 16, scale_body, 0)
            pltpu.sync_copy(rows, acc.at[dstadj], add=True)
            return 0
        lax.fori_loop(0, BPT, block_body, 0)
        plsc.subcore_barrier()

        for c in range(NCH):
            lbase = wid * HPT + c * HCH
            gbase = lo + lbase
            pltpu.sync_copy(acc.at[pl.ds(lbase, HCH)],
                            rows.at[pl.ds(0, HCH)])
            pltpu.sync_copy(x_hbm.at[pl.ds(gbase, HCH)], nbuf)

            def node_body(g, _):
                dv16 = dinvb[pl.ds(gbase + g * 16, 16)]
                for k in range(16):
                    dv = dv16[k]
                    dv2 = dv * dv * slw
                    nn = g * 16 + k
                    for j in range(D // 16):
                        sl = pl.ds(j * 16, 16)
                        rows[nn, sl] = rows[nn, sl] + dv2 * nbuf[nn, sl]
                return 0
            lax.fori_loop(0, HCH // 16, node_body, 0)

            @pl.when(cid == 0)
            def _():
                pltpu.sync_copy(rows.at[pl.ds(0, HCH)],
                                o0_hbm.at[pl.ds(gbase, HCH)])

            @pl.when(cid == 1)
            def _():
                pltpu.sync_copy(rows.at[pl.ds(0, HCH)],
                                o1_hbm.at[pl.ds(gbase, HCH)])
        plsc.subcore_barrier()
        return 0
    lax.fori_loop(0, NPASS, pass_body, 0)


@functools.partial(
    pl.kernel, mesh=_MESH, compiler_params=_CPARAMS,
    out_type=(jax.ShapeDtypeStruct((2 * POS, D), jnp.float32),
              jax.ShapeDtypeStruct((2 * POS, D), jnp.float32)),
    scratch_types=[
        pltpu.VMEM((2 * POS,), jnp.int32),
        pltpu.VMEM((2 * POS, D), jnp.float32),
        pltpu.SemaphoreType.DMA,
    ],
)
def _gather_pos_kernel(a0_hbm, a1_hbm, pos_hbm, g0_hbm, g1_hbm,
                       posb, buf, sem):
    cid = lax.axis_index("c")
    wid = lax.axis_index("s")

    @pl.when((cid == 0) & (wid == 0))
    def _():
        pltpu.sync_copy(pos_hbm, posb)
        pltpu.async_copy(a0_hbm.at[posb], buf, sem).wait()
        pltpu.sync_copy(buf, g0_hbm)
        pltpu.async_copy(a1_hbm.at[posb], buf, sem).wait()
        pltpu.sync_copy(buf, g1_hbm)


def _final_mm_tc(pg0, pg1, w, b):
    def body(x0_ref, x1_ref, w_ref, b_ref, o_ref):
        o_ref[...] = jnp.dot(x0_ref[...] + x1_ref[...], w_ref[...],
                             preferred_element_type=jnp.float32) + b_ref[...]
    return pl.pallas_call(
        body,
        out_shape=jax.ShapeDtypeStruct((2 * POS, D), jnp.float32),
    )(pg0, pg1, w, b.reshape(1, D))


@jax.jit
def kernel(x, edge_index, edge_weight, pos, W1, b1, W2, b2):
    pad = PE - E
    comb = edge_index[0] | (edge_index[1] << 14)
    combp = jnp.concatenate(
        [comb, jnp.zeros((pad,), jnp.int32)]).reshape(NBLK, EB)
    wp = jnp.concatenate(
        [edge_weight, jnp.zeros((pad,), jnp.float32)]).reshape(NBLK, EB)

    deg_parts = _deg_kernel(combp, wp)
    dinv = _dinv_tc(deg_parts.reshape(NW, NP // D, D)).reshape(NP)

    xp = jnp.concatenate([x, jnp.zeros((NP - N, D), jnp.float32)])
    a1_0, a1_1 = _agg_kernel(combp, wp, dinv, xp)
    h = _mm_bias_tc(a1_0, a1_1, W1, b1, relu=True, bm=512)
    a2_0, a2_1 = _agg_kernel(combp, wp, dinv, h)

    posp = jnp.concatenate([pos, jnp.zeros((POS,), jnp.int32)])
    pg0, pg1 = _gather_pos_kernel(a2_0, a2_1, posp)
    out16 = _final_mm_tc(pg0, pg1, W2, b2)

    out = jnp.where((pos != -1)[:, None], out16[:POS], jnp.float32(-3.0))
    return out.reshape(1, POS * D)
